# baseline (device time: 18288 ns/iter reference)
import jax
import jax.numpy as jnp
from jax import lax
from jax.experimental import pallas as pl
from jax.experimental.pallas import tpu as pltpu

N_DEV = 4

QCLIP = 5.0
INV_SCALE = 127.0 / QCLIP
DEQ_SCALE = QCLIP / 127.0

_ORDER = (0, 3, 2, 1)


def kernel(x, w_mat):
    m_total, k_shard = x.shape
    k_total, n = w_mat.shape
    m_per = m_total // N_DEV

    def body(x_hbm, w_hbm, out_hbm,
             xv, q_stage, q_comm, xbf, w_full, acc,
             send_sems, recv_sems, w_sems, x_sems, out_sems):
        my = lax.axis_index("i")

        x_cps = {}
        for idx, d in enumerate((1, 2, 3, 0)):
            j = lax.rem(my + d, N_DEV)
            cp = pltpu.make_async_copy(
                x_hbm.at[pl.ds(j * m_per, m_per), :],
                xv.at[pl.ds(j * m_per, m_per), :],
                x_sems.at[idx],
            )
            cp.start()
            x_cps[d] = cp

        def w_fetch(idx):
            j = lax.rem(my + _ORDER[idx], N_DEV)
            cp = pltpu.make_async_copy(
                w_hbm.at[pl.ds(j * k_shard, k_shard), :],
                w_full.at[pl.ds(j * k_shard, k_shard), :],
                w_sems.at[idx],
            )
            cp.start()
            return cp

        w_cps = [w_fetch(0), w_fetch(1)]

        barrier_sem = pltpu.get_barrier_semaphore()
        for d in range(1, N_DEV):
            peer = lax.rem(my + d, N_DEV)
            pl.semaphore_signal(
                barrier_sem, inc=1,
                device_id=(peer,), device_id_type=pl.DeviceIdType.MESH,
            )

        def quantize(d):
            peer = lax.rem(my + d, N_DEV)
            x_cps[d].wait()
            blk = xv[pl.ds(peer * m_per, m_per), :]
            q_stage[d] = jnp.clip(
                jnp.round(blk * INV_SCALE), -127.0, 127.0
            ).astype(jnp.int8)

        quantize(1)
        pl.semaphore_wait(barrier_sem, N_DEV - 1)

        sends = []
        for d in range(1, N_DEV):
            peer = lax.rem(my + d, N_DEV)
            rdma = pltpu.make_async_remote_copy(
                src_ref=q_stage.at[d],
                dst_ref=q_comm.at[my],
                send_sem=send_sems.at[d - 1],
                recv_sem=recv_sems.at[my],
                device_id=(peer,),
                device_id_type=pl.DeviceIdType.MESH,
            )
            rdma.start()
            sends.append(rdma)
            if d < N_DEV - 1:
                quantize(d + 1)

        x_cps[0].wait()
        xbf[:, pl.ds(my * k_shard, k_shard)] = (
            xv[pl.ds(my * m_per, m_per), :] * INV_SCALE
        ).astype(jnp.bfloat16)

        def dequant(d):
            src = lax.rem(my + d, N_DEV)
            recv = pltpu.make_async_remote_copy(
                src_ref=q_comm.at[src],
                dst_ref=q_comm.at[src],
                send_sem=send_sems.at[d - 1],
                recv_sem=recv_sems.at[src],
                device_id=(src,),
                device_id_type=pl.DeviceIdType.MESH,
            )
            recv.wait_recv()
            xbf[:, pl.ds(src * k_shard, k_shard)] = q_comm[src].astype(
                jnp.bfloat16
            )
            return src

        out_cps = []
        for idx, d in enumerate(_ORDER):
            src = lax.rem(my + d, N_DEV) if d == 0 else dequant(d)
            w_cps[idx].wait()
            if idx + 2 < N_DEV:
                w_cps.append(w_fetch(idx + 2))
            xk = xbf[:, pl.ds(src * k_shard, k_shard)]
            if idx == 0:
                acc[...] = jnp.dot(
                    xk, w_full[pl.ds(src * k_shard, k_shard), :],
                    preferred_element_type=jnp.float32,
                )
            elif idx < N_DEV - 1:
                acc[...] += jnp.dot(
                    xk, w_full[pl.ds(src * k_shard, k_shard), :],
                    preferred_element_type=jnp.float32,
                )
            else:
                nq = n // 4
                for h in range(4):
                    part = jnp.dot(
                        xk,
                        w_full[pl.ds(src * k_shard, k_shard),
                               h * nq:(h + 1) * nq],
                        preferred_element_type=jnp.float32,
                    )
                    acc[:, h * nq:(h + 1) * nq] = (
                        jnp.maximum(acc[:, h * nq:(h + 1) * nq] + part, 0.0)
                        * DEQ_SCALE
                    )
                    cp = pltpu.make_async_copy(
                        acc.at[:, h * nq:(h + 1) * nq],
                        out_hbm.at[:, h * nq:(h + 1) * nq],
                        out_sems.at[h],
                    )
                    cp.start()
                    out_cps.append(cp)

        for cp in out_cps:
            cp.wait()
        for rdma in sends:
            rdma.wait_send()

    return pl.pallas_call(
        body,
        out_shape=jax.ShapeDtypeStruct((m_per, n), jnp.float32),
        in_specs=[
            pl.BlockSpec(memory_space=pl.ANY),
            pl.BlockSpec(memory_space=pl.ANY),
        ],
        out_specs=pl.BlockSpec(memory_space=pl.ANY),
        scratch_shapes=[
            pltpu.VMEM((m_total, k_shard), jnp.float32),
            pltpu.VMEM((N_DEV, m_per, k_shard), jnp.int8),
            pltpu.VMEM((N_DEV, m_per, k_shard), jnp.int8),
            pltpu.VMEM((m_per, k_total), jnp.bfloat16),
            pltpu.VMEM((k_total, n), jnp.float32),
            pltpu.VMEM((m_per, n), jnp.float32),
            pltpu.SemaphoreType.DMA((N_DEV - 1,)),
            pltpu.SemaphoreType.DMA((N_DEV,)),
            pltpu.SemaphoreType.DMA((N_DEV,)),
            pltpu.SemaphoreType.DMA((N_DEV,)),
            pltpu.SemaphoreType.DMA((4,)),
        ],
        compiler_params=pltpu.CompilerParams(collective_id=0),
    )(x, w_mat)


# device time: 16736 ns/iter; 1.0927x vs baseline; 1.0927x over previous
import jax
import jax.numpy as jnp
from jax import lax
from jax.experimental import pallas as pl
from jax.experimental.pallas import tpu as pltpu

N_DEV = 4

QCLIP = 5.0
INV_SCALE = 127.0 / QCLIP
DEQ_SCALE = QCLIP / 127.0

_ORDER = (0, 3, 2, 1)


def kernel(x, w_mat):
    m_total, k_shard = x.shape
    k_total, n = w_mat.shape
    m_per = m_total // N_DEV

    def body(x_hbm, w_hbm, out_ref,
             xv, q_stage, q_comm, xbf, w_full,
             send_sems, recv_sems, w_sems, x_sems):
        my = lax.axis_index("i")

        x_cps = {}
        for idx, d in enumerate((1, 2, 3, 0)):
            j = lax.rem(my + d, N_DEV)
            cp = pltpu.make_async_copy(
                x_hbm.at[pl.ds(j * m_per, m_per), :],
                xv.at[pl.ds(j * m_per, m_per), :],
                x_sems.at[idx],
            )
            cp.start()
            x_cps[d] = cp

        def w_fetch(idx):
            j = lax.rem(my + _ORDER[idx], N_DEV)
            cp = pltpu.make_async_copy(
                w_hbm.at[pl.ds(j * k_shard, k_shard), :],
                w_full.at[pl.ds(j * k_shard, k_shard), :],
                w_sems.at[idx],
            )
            cp.start()
            return cp

        w_cps = [w_fetch(0), w_fetch(1)]

        barrier_sem = pltpu.get_barrier_semaphore()
        for d in range(1, N_DEV):
            peer = lax.rem(my + d, N_DEV)
            pl.semaphore_signal(
                barrier_sem, inc=1,
                device_id=(peer,), device_id_type=pl.DeviceIdType.MESH,
            )

        def quantize(d):
            peer = lax.rem(my + d, N_DEV)
            x_cps[d].wait()
            blk = xv[pl.ds(peer * m_per, m_per), :]
            q_stage[d] = jnp.clip(
                jnp.round(blk * INV_SCALE), -127.0, 127.0
            ).astype(jnp.int8)

        quantize(1)
        pl.semaphore_wait(barrier_sem, N_DEV - 1)

        sends = []
        for d in range(1, N_DEV):
            peer = lax.rem(my + d, N_DEV)
            rdma = pltpu.make_async_remote_copy(
                src_ref=q_stage.at[d],
                dst_ref=q_comm.at[my],
                send_sem=send_sems.at[d - 1],
                recv_sem=recv_sems.at[my],
                device_id=(peer,),
                device_id_type=pl.DeviceIdType.MESH,
            )
            rdma.start()
            sends.append(rdma)
            if d < N_DEV - 1:
                quantize(d + 1)

        x_cps[0].wait()
        xbf[:, pl.ds(my * k_shard, k_shard)] = (
            xv[pl.ds(my * m_per, m_per), :] * INV_SCALE
        ).astype(jnp.bfloat16)

        def dequant(d):
            src = lax.rem(my + d, N_DEV)
            recv = pltpu.make_async_remote_copy(
                src_ref=q_comm.at[src],
                dst_ref=q_comm.at[src],
                send_sem=send_sems.at[d - 1],
                recv_sem=recv_sems.at[src],
                device_id=(src,),
                device_id_type=pl.DeviceIdType.MESH,
            )
            recv.wait_recv()
            xbf[:, pl.ds(src * k_shard, k_shard)] = q_comm[src].astype(
                jnp.bfloat16
            )
            return src

        for idx, d in enumerate(_ORDER):
            src = lax.rem(my + d, N_DEV) if d == 0 else dequant(d)
            w_cps[idx].wait()
            if idx + 2 < N_DEV:
                w_cps.append(w_fetch(idx + 2))
            part = jnp.dot(
                xbf[:, pl.ds(src * k_shard, k_shard)],
                w_full[pl.ds(src * k_shard, k_shard), :],
                preferred_element_type=jnp.float32,
            )
            if idx == 0:
                out_ref[...] = part
            elif idx < N_DEV - 1:
                out_ref[...] += part
            else:
                out_ref[...] = (
                    jnp.maximum(out_ref[...] + part, 0.0) * DEQ_SCALE
                )

        for rdma in sends:
            rdma.wait_send()

    return pl.pallas_call(
        body,
        out_shape=jax.ShapeDtypeStruct((m_per, n), jnp.float32),
        in_specs=[
            pl.BlockSpec(memory_space=pl.ANY),
            pl.BlockSpec(memory_space=pl.ANY),
        ],
        out_specs=pl.BlockSpec(memory_space=pltpu.VMEM),
        scratch_shapes=[
            pltpu.VMEM((m_total, k_shard), jnp.float32),
            pltpu.VMEM((N_DEV, m_per, k_shard), jnp.int8),
            pltpu.VMEM((N_DEV, m_per, k_shard), jnp.int8),
            pltpu.VMEM((m_per, k_total), jnp.bfloat16),
            pltpu.VMEM((k_total, n), jnp.float32),
            pltpu.SemaphoreType.DMA((N_DEV - 1,)),
            pltpu.SemaphoreType.DMA((N_DEV,)),
            pltpu.SemaphoreType.DMA((N_DEV,)),
            pltpu.SemaphoreType.DMA((N_DEV,)),
        ],
        compiler_params=pltpu.CompilerParams(collective_id=0),
    )(x, w_mat)
